# baseline (speedup 1.0000x reference)
import functools

import jax
import jax.numpy as jnp
from jax import lax
from jax.experimental import pallas as pl
from jax.experimental.pallas import tpu as pltpu
from jax.experimental.pallas import tpu_sc as plsc

N_NODES = 1000000
DIM = 16
BATCH = 16384
L = 16

NC = 2
NS = 16
NW = NC * NS
ROWS_PER_W = N_NODES // NW
T_SIZE = ((ROWS_PER_W + L - 1) // L) * L
CHUNK = 128

_mesh = plsc.VectorSubcoreMesh(core_axis_name="c", subcore_axis_name="s")


@functools.partial(
    pl.kernel,
    mesh=_mesh,
    compiler_params=pltpu.CompilerParams(
        needs_layout_passes=False, use_tc_tiling_on_sc=False),
    scratch_types=[
        pltpu.VMEM((BATCH,), jnp.int32),
        pltpu.VMEM((T_SIZE,), jnp.int32),
        pltpu.VMEM((BATCH,), jnp.int32),
        pltpu.VMEM((BATCH,), jnp.int32),
        pltpu.VMEM((CHUNK,), jnp.int32),
        pltpu.VMEM((CHUNK,), jnp.int32),
        pltpu.VMEM((DIM, CHUNK), jnp.float32),
        pltpu.SemaphoreType.DMA,
        pltpu.SemaphoreType.DMA,
    ],
)
def _sc_scatter(nids_hbm, valt_hbm, outt_hbm,
                nids_v, t_v, wb_v, wn_v, idxb_c, idxn_c, cols_v,
                sem_g, sem_s):
    wid = lax.axis_index("s") * NC + lax.axis_index("c")
    base = wid * ROWS_PER_W
    iota = lax.iota(jnp.int32, L)
    neg1 = jnp.full((L,), -1, jnp.int32)

    pltpu.sync_copy(nids_hbm, nids_v)

    def init_body(i, carry):
        t_v[pl.ds(i * L, L)] = neg1
        return carry
    lax.fori_loop(0, T_SIZE // L, init_body, 0, unroll=4)

    def stamp_body(i, carry):
        v = nids_v[pl.ds(i * L, L)]
        inr = (v >= base) & (v < base + ROWS_PER_W)
        _, last = plsc.scan_count(v, mask=inr)
        m = inr & last
        local = jnp.where(m, v - base, 0)
        bidx = iota + i * L
        plsc.store_scatter(t_v, [local], bidx, mask=m)
        return carry
    lax.fori_loop(0, BATCH // L, stamp_body, 0, unroll=2)

    def compact_body(k, cnt):
        t = t_v[pl.ds(k * L, L)]
        m = t >= 0
        m_i32 = m.astype(jnp.int32)
        inc = plsc.cumsum(m_i32)
        pos = cnt + inc - m_i32
        nvec = base + k * L + iota
        plsc.store_scatter(wb_v, [pos], t, mask=m)
        plsc.store_scatter(wn_v, [pos], nvec, mask=m)
        return cnt + jnp.max(inc)
    cnt = lax.fori_loop(0, T_SIZE // L, compact_body, jnp.int32(0), unroll=2)

    @pl.when(cnt > 0)
    def _tail():
        nchunks = (cnt + CHUNK - 1) // CHUNK
        cnt_pad = nchunks * CHUNK
        aligned = (cnt - 1) & ~(L - 1)
        vb = wb_v[pl.ds(aligned, L)]
        vn = wn_v[pl.ds(aligned, L)]
        lane = cnt - 1 - aligned
        b_last = jnp.max(jnp.where(iota == lane, vb, jnp.int32(-2147483648)))
        n_last = jnp.max(jnp.where(iota == lane, vn, jnp.int32(-2147483648)))
        b_splat = jnp.full((L,), 0, jnp.int32) + b_last
        n_splat = jnp.full((L,), 0, jnp.int32) + n_last

        def pad_body(p, carry):
            pvec = p * L + iota
            fm = pvec >= cnt
            plsc.store_scatter(wb_v, [pvec], b_splat, mask=fm)
            plsc.store_scatter(wn_v, [pvec], n_splat, mask=fm)
            return carry
        lax.fori_loop(aligned // L, cnt_pad // L, pad_body, 0)

        def chunk_body(c, carry):
            for j in range(CHUNK // L):
                off = c * CHUNK + j * L
                idxb_c[pl.ds(j * L, L)] = wb_v[pl.ds(off, L)]
                idxn_c[pl.ds(j * L, L)] = wn_v[pl.ds(off, L)]
            gathers = [
                pltpu.async_copy(valt_hbm.at[d].at[idxb_c], cols_v.at[d],
                                 sem_g)
                for d in range(DIM)
            ]
            for g in gathers:
                g.wait()
            scatters = [
                pltpu.async_copy(cols_v.at[d], outt_hbm.at[d].at[idxn_c],
                                 sem_s)
                for d in range(DIM)
            ]
            for s in scatters:
                s.wait()
            return carry
        lax.fori_loop(0, nchunks, chunk_body, 0)


def kernel(memory, nids, val):
    outt = jax.new_ref(memory.T)
    _sc_scatter(nids, val.T, outt)
    return outt[...].T


# speedup vs baseline: 4.8168x; 4.8168x over previous
import functools

import jax
import jax.numpy as jnp
from jax import lax
from jax.experimental import pallas as pl
from jax.experimental.pallas import tpu as pltpu
from jax.experimental.pallas import tpu_sc as plsc

N_NODES = 1000000
DIM = 16
BATCH = 16384
L = 16

NC = 2
NS = 16
NW = NC * NS
ROWS_PER_W = N_NODES // NW
T_SIZE = ((ROWS_PER_W + L - 1) // L) * L
CHUNK = 128

LIN_ROWS = N_NODES * DIM // 128


C_BLK = 8192
_TGRID = (N_NODES + C_BLK - 1) // C_BLK


def _to_rows_body(x_ref, o_ref, s1, s2):
    s1[...] = x_ref[...].T
    s2[...] = jnp.reshape(s1[...], (C_BLK // 8, 8, DIM))
    o_ref[...] = jnp.reshape(s2[...], (C_BLK // 8, 128))


_to_rows = pl.pallas_call(
    _to_rows_body,
    grid=(_TGRID,),
    in_specs=[pl.BlockSpec((DIM, C_BLK), lambda j: (0, j))],
    out_specs=pl.BlockSpec((C_BLK // 8, 128), lambda j: (j, 0)),
    out_shape=jax.ShapeDtypeStruct((LIN_ROWS, 128), jnp.float32),
    scratch_shapes=[
        pltpu.VMEM((C_BLK, DIM), jnp.float32),
        pltpu.VMEM((C_BLK // 8, 8, DIM), jnp.float32),
    ],
)


def _from_rows_body(z_ref, o_ref, s1, s2):
    s2[...] = jnp.reshape(z_ref[...], (C_BLK // 8, 8, DIM))
    s1[...] = jnp.reshape(s2[...], (C_BLK, DIM))
    o_ref[...] = s1[...].T


_from_rows = pl.pallas_call(
    _from_rows_body,
    grid=(_TGRID,),
    in_specs=[pl.BlockSpec((C_BLK // 8, 128), lambda j: (j, 0))],
    out_specs=pl.BlockSpec((DIM, C_BLK), lambda j: (0, j)),
    out_shape=jax.ShapeDtypeStruct((DIM, N_NODES), jnp.float32),
    scratch_shapes=[
        pltpu.VMEM((C_BLK, DIM), jnp.float32),
        pltpu.VMEM((C_BLK // 8, 8, DIM), jnp.float32),
    ],
)


_mesh = plsc.VectorSubcoreMesh(core_axis_name="c", subcore_axis_name="s")


@functools.partial(
    pl.kernel,
    mesh=_mesh,
    compiler_params=pltpu.CompilerParams(
        needs_layout_passes=False, use_tc_tiling_on_sc=False),
    scratch_types=[
        pltpu.VMEM((BATCH,), jnp.int32),
        pltpu.VMEM((T_SIZE,), jnp.int32),
        pltpu.VMEM((BATCH,), jnp.int32),
        pltpu.VMEM((BATCH,), jnp.int32),
        pltpu.VMEM((CHUNK,), jnp.int32),
        pltpu.VMEM((CHUNK,), jnp.int32),
        pltpu.VMEM((CHUNK, DIM), jnp.float32),
        pltpu.SemaphoreType.DMA,
        pltpu.SemaphoreType.DMA,
    ],
)
def _sc_scatter(nids_hbm, val_hbm, out_hbm,
                nids_v, t_v, wb_v, wn_v, idxb_c, idxn_c, rows_v,
                sem_g, sem_s):
    wid = lax.axis_index("s") * NC + lax.axis_index("c")
    base = wid * ROWS_PER_W
    iota = lax.iota(jnp.int32, L)
    neg1 = jnp.full((L,), -1, jnp.int32)

    pltpu.sync_copy(nids_hbm, nids_v)

    def init_body(i, carry):
        t_v[pl.ds(i * L, L)] = neg1
        return carry
    lax.fori_loop(0, T_SIZE // L, init_body, 0, unroll=4)

    def stamp_body(i, carry):
        v = nids_v[pl.ds(i * L, L)]
        inr = (v >= base) & (v < base + ROWS_PER_W)
        _, last = plsc.scan_count(v, mask=inr)
        m = inr & last
        local = jnp.where(m, v - base, 0)
        bidx = iota + i * L
        plsc.store_scatter(t_v, [local], bidx, mask=m)
        return carry
    lax.fori_loop(0, BATCH // L, stamp_body, 0, unroll=2)

    def compact_body(k, cnt):
        t = t_v[pl.ds(k * L, L)]
        m = t >= 0
        m_i32 = m.astype(jnp.int32)
        inc = plsc.cumsum(m_i32)
        pos = cnt + inc - m_i32
        nvec = base + k * L + iota
        plsc.store_scatter(wb_v, [pos], t, mask=m)
        plsc.store_scatter(wn_v, [pos], nvec, mask=m)
        return cnt + jnp.max(inc)
    cnt = lax.fori_loop(0, T_SIZE // L, compact_body, jnp.int32(0), unroll=2)

    @pl.when(cnt > 0)
    def _tail():
        nchunks = (cnt + CHUNK - 1) // CHUNK
        cnt_pad = nchunks * CHUNK
        aligned = (cnt - 1) & ~(L - 1)
        vb = wb_v[pl.ds(aligned, L)]
        vn = wn_v[pl.ds(aligned, L)]
        lane = cnt - 1 - aligned
        b_last = jnp.max(jnp.where(iota == lane, vb, jnp.int32(-2147483648)))
        n_last = jnp.max(jnp.where(iota == lane, vn, jnp.int32(-2147483648)))
        b_splat = jnp.full((L,), 0, jnp.int32) + b_last
        n_splat = jnp.full((L,), 0, jnp.int32) + n_last

        def pad_body(p, carry):
            pvec = p * L + iota
            fm = pvec >= cnt
            plsc.store_scatter(wb_v, [pvec], b_splat, mask=fm)
            plsc.store_scatter(wn_v, [pvec], n_splat, mask=fm)
            return carry
        lax.fori_loop(aligned // L, cnt_pad // L, pad_body, 0)

        def chunk_body(c, carry):
            for j in range(CHUNK // L):
                off = c * CHUNK + j * L
                idxb_c[pl.ds(j * L, L)] = wb_v[pl.ds(off, L)]
                idxn_c[pl.ds(j * L, L)] = wn_v[pl.ds(off, L)]
            pltpu.async_copy(val_hbm.at[idxb_c], rows_v, sem_g).wait()
            pltpu.async_copy(rows_v, out_hbm.at[idxn_c], sem_s).wait()
            return carry
        lax.fori_loop(0, nchunks, chunk_body, 0)


def kernel(memory, nids, val):
    y = _to_rows(memory.T)
    y_rows = jnp.reshape(y, (N_NODES, DIM))
    r = jax.new_ref(y_rows)
    _sc_scatter(nids, val, r)
    z = jnp.reshape(r[...], (LIN_ROWS, 128))
    return _from_rows(z).T


# speedup vs baseline: 5.4770x; 1.1371x over previous
import functools

import jax
import jax.numpy as jnp
from jax import lax
from jax.experimental import pallas as pl
from jax.experimental.pallas import tpu as pltpu
from jax.experimental.pallas import tpu_sc as plsc

N_NODES = 1000000
DIM = 16
BATCH = 16384
L = 16

NC = 2
NS = 16
NW = NC * NS
ROWS_PER_W = N_NODES // NW
T_SIZE = ((ROWS_PER_W + L - 1) // L) * L
CHUNK = 128
SUB = CHUNK // L

NTC = (N_NODES + 127) // 128
FLAT = NTC * DIM * 128


TCG = 64
C_BLK = TCG * 128
_TGRID = (N_NODES + C_BLK - 1) // C_BLK


def _to_tiles_body(x_ref, o_ref):
    x = x_ref[...]
    o_ref[...] = jnp.transpose(jnp.reshape(x, (DIM, TCG, 128)), (1, 0, 2))


_to_tiles = pl.pallas_call(
    _to_tiles_body,
    grid=(_TGRID,),
    in_specs=[pl.BlockSpec((DIM, C_BLK), lambda j: (0, j))],
    out_specs=pl.BlockSpec((TCG, DIM, 128), lambda j: (j, 0, 0)),
    out_shape=jax.ShapeDtypeStruct((NTC, DIM, 128), jnp.float32),
)


def _from_tiles_body(z_ref, o_ref):
    z = z_ref[...]
    o_ref[...] = jnp.reshape(jnp.transpose(z, (1, 0, 2)), (DIM, C_BLK))


_from_tiles = pl.pallas_call(
    _from_tiles_body,
    grid=(_TGRID,),
    in_specs=[pl.BlockSpec((TCG, DIM, 128), lambda j: (j, 0, 0))],
    out_specs=pl.BlockSpec((DIM, C_BLK), lambda j: (0, j)),
    out_shape=jax.ShapeDtypeStruct((DIM, N_NODES), jnp.float32),
)


_mesh = plsc.VectorSubcoreMesh(core_axis_name="c", subcore_axis_name="s")


@functools.partial(
    pl.kernel,
    mesh=_mesh,
    compiler_params=pltpu.CompilerParams(
        needs_layout_passes=False, use_tc_tiling_on_sc=False),
    scratch_types=[
        pltpu.VMEM((BATCH,), jnp.int32),
        pltpu.VMEM((T_SIZE,), jnp.int32),
        pltpu.VMEM((BATCH,), jnp.int32),
        pltpu.VMEM((BATCH,), jnp.int32),
        pltpu.VMEM((CHUNK,), jnp.int32),
        pltpu.VMEM((CHUNK, DIM), jnp.float32),
        pltpu.VMEM((2 * SUB, CHUNK), jnp.int32),
        pltpu.VMEM((2 * SUB, CHUNK), jnp.float32),
        pltpu.SemaphoreType.DMA,
        pltpu.SemaphoreType.DMA,
    ],
)
def _sc_patch(nids_hbm, val_hbm, out_hbm,
              nids_v, t_v, wb_v, wn_v, idxb_c, rows_v, idxe_c, srce_c,
              sem_g, sem_s):
    wid = lax.axis_index("s") * NC + lax.axis_index("c")
    base = wid * ROWS_PER_W
    iota = lax.iota(jnp.int32, L)
    neg1 = jnp.full((L,), -1, jnp.int32)

    pltpu.sync_copy(nids_hbm, nids_v)

    def init_body(i, carry):
        t_v[pl.ds(i * L, L)] = neg1
        return carry
    lax.fori_loop(0, T_SIZE // L, init_body, 0, unroll=4)

    def stamp_body(i, carry):
        v = nids_v[pl.ds(i * L, L)]
        inr = (v >= base) & (v < base + ROWS_PER_W)
        _, last = plsc.scan_count(v, mask=inr)
        m = inr & last
        local = jnp.where(m, v - base, 0)
        bidx = iota + i * L
        plsc.store_scatter(t_v, [local], bidx, mask=m)
        return carry
    lax.fori_loop(0, BATCH // L, stamp_body, 0, unroll=2)

    def compact_body(k, cnt):
        t = t_v[pl.ds(k * L, L)]
        m = t >= 0
        m_i32 = m.astype(jnp.int32)
        inc = plsc.cumsum(m_i32)
        pos = cnt + inc - m_i32
        nvec = base + k * L + iota
        plsc.store_scatter(wb_v, [pos], t, mask=m)
        plsc.store_scatter(wn_v, [pos], nvec, mask=m)
        return cnt + jnp.max(inc)
    cnt = lax.fori_loop(0, T_SIZE // L, compact_body, jnp.int32(0), unroll=2)

    @pl.when(cnt > 0)
    def _tail():
        nchunks = (cnt + CHUNK - 1) // CHUNK
        cnt_pad = nchunks * CHUNK
        aligned = (cnt - 1) & ~(L - 1)
        vb = wb_v[pl.ds(aligned, L)]
        vn = wn_v[pl.ds(aligned, L)]
        lane = cnt - 1 - aligned
        b_last = jnp.max(jnp.where(iota == lane, vb, jnp.int32(-2147483648)))
        n_last = jnp.max(jnp.where(iota == lane, vn, jnp.int32(-2147483648)))
        b_splat = jnp.full((L,), 0, jnp.int32) + b_last
        n_splat = jnp.full((L,), 0, jnp.int32) + n_last

        def pad_body(p, carry):
            pvec = p * L + iota
            fm = pvec >= cnt
            plsc.store_scatter(wb_v, [pvec], b_splat, mask=fm)
            plsc.store_scatter(wn_v, [pvec], n_splat, mask=fm)
            return carry
        lax.fori_loop(aligned // L, cnt_pad // L, pad_body, 0)

        def chunk_body(c, carry):
            for j in range(CHUNK // L):
                off = c * CHUNK + j * L
                idxb_c[pl.ds(j * L, L)] = wb_v[pl.ds(off, L)]
            pltpu.async_copy(val_hbm.at[idxb_c], rows_v, sem_g).wait()

            copies = []
            for s in range(CHUNK // L):
                nv = wn_v[pl.ds(c * CHUNK + s * L, L)]
                bases = (nv // 128) * (DIM * 128) + (nv % 128)
                for j in range(L):
                    bj = jnp.max(jnp.where(iota == j, bases,
                                           jnp.int32(-2147483648)))
                    row = plsc.load_gather(
                        rows_v, [jnp.full((L,), s * L + j, jnp.int32), iota])
                    buf = 2 * s + j // SUB
                    slot = j % SUB
                    idxe_c[buf, pl.ds(slot * L, L)] = bj + iota * 128
                    srce_c[buf, pl.ds(slot * L, L)] = row
                copies.append(pltpu.async_copy(
                    srce_c.at[2 * s], out_hbm.at[idxe_c.at[2 * s]], sem_s))
                copies.append(pltpu.async_copy(
                    srce_c.at[2 * s + 1],
                    out_hbm.at[idxe_c.at[2 * s + 1]], sem_s))
            for cp in copies:
                cp.wait()
            return carry
        lax.fori_loop(0, nchunks, chunk_body, 0)


def kernel(memory, nids, val):
    yt = _to_tiles(memory.T)
    r = jax.new_ref(jnp.reshape(yt, (FLAT,)))
    _sc_patch(nids, val, r)
    z = jnp.reshape(r[...], (NTC, DIM, 128))
    return _from_tiles(z).T
